# baseline (speedup 1.0000x reference)
import functools
import numpy as np
import jax
import jax.numpy as jnp
from jax.experimental import pallas as pl

T = 2048
D = 1024
H = 16
DH = 64
K_ACT = 4
ROPE_BASE = 10000.0
BT = 256
BQ = 256
BKV = 256
NEG = -1e30

_dot = functools.partial(jax.lax.dot_general, preferred_element_type=jnp.float32)


def _proj_kernel(x_ref, wq_ref, wk_ref, wv_ref, wr_ref, cs_ref, q_ref, k_ref, v_ref, g_ref):
    x = x_ref[...]
    q = _dot(x, wq_ref[...], (((1,), (0,)), ((), ())))
    k = _dot(x, wk_ref[...], (((1,), (0,)), ((), ())))
    v_ref[...] = _dot(x, wv_ref[...], (((1,), (0,)), ((), ())))

    c = cs_ref[:, 0:32]
    s = cs_ref[:, 32:64]
    q_parts = []
    k_parts = []
    for h in range(H):
        q1 = q[:, h * DH:h * DH + 32]
        q2 = q[:, h * DH + 32:(h + 1) * DH]
        k1 = k[:, h * DH:h * DH + 32]
        k2 = k[:, h * DH + 32:(h + 1) * DH]
        q_parts.append(q1 * c - q2 * s)
        q_parts.append(q1 * s + q2 * c)
        k_parts.append(k1 * c - k2 * s)
        k_parts.append(k1 * s + k2 * c)
    q_ref[...] = jnp.concatenate(q_parts, axis=1)
    k_ref[...] = jnp.concatenate(k_parts, axis=1)

    logits = _dot(x, wr_ref[...], (((1,), (0,)), ((), ())))
    lane = jax.lax.broadcasted_iota(jnp.int32, (BT, 128), 1)
    valid = lane < H
    logits = jnp.where(valid, logits, NEG)
    m = jnp.max(logits, axis=-1, keepdims=True)
    e = jnp.where(valid, jnp.exp(logits - m), 0.0)
    p = e / jnp.sum(e, axis=-1, keepdims=True)

    avail = valid
    gates = jnp.zeros_like(p)
    tot = jnp.zeros((BT, 1), dtype=jnp.float32)
    for _ in range(K_ACT):
        pm = jnp.where(avail, p, -1.0)
        mx = jnp.max(pm, axis=-1, keepdims=True)
        first = jnp.min(jnp.where(pm == mx, lane, 9999), axis=-1, keepdims=True)
        sel = lane == first
        gates = gates + jnp.where(sel, p, 0.0)
        tot = tot + mx
        avail = jnp.logical_and(avail, jnp.logical_not(sel))
    g_ref[...] = gates / (tot + 1e-9)


def _attn_kernel(q_ref, k_ref, v_ref, g_ref, y_ref):
    i = pl.program_id(1)
    hp = pl.program_id(0)
    scale = jnp.float32(1.0 / np.sqrt(DH))
    row = jax.lax.broadcasted_iota(jnp.int32, (BQ, BKV), 0) + i * BQ
    col = jax.lax.broadcasted_iota(jnp.int32, (BQ, BKV), 1)
    lane = jax.lax.broadcasted_iota(jnp.int32, (BQ, 128), 1)
    g = g_ref[...]
    nkv = (i * BQ) // BKV + 1
    for sub in range(2):
        q = q_ref[:, sub * DH:(sub + 1) * DH]

        def body(j, carry):
            acc, m, l = carry
            k = k_ref[pl.ds(j * BKV, BKV), sub * DH:(sub + 1) * DH]
            v = v_ref[pl.ds(j * BKV, BKV), sub * DH:(sub + 1) * DH]
            s = _dot(q, k, (((1,), (1,)), ((), ()))) * scale
            s = jnp.where(col + j * BKV <= row, s, NEG)
            m_new = jnp.maximum(m, jnp.max(s, axis=-1, keepdims=True))
            p = jnp.exp(s - m_new)
            alpha = jnp.exp(m - m_new)
            l = l * alpha + jnp.sum(p, axis=-1, keepdims=True)
            acc = acc * alpha + _dot(p, v, (((1,), (0,)), ((), ())))
            return acc, m_new, l

        acc, m, l = jax.lax.fori_loop(
            0, nkv, body,
            (jnp.zeros((BQ, DH), jnp.float32),
             jnp.full((BQ, 1), NEG, jnp.float32),
             jnp.zeros((BQ, 1), jnp.float32)))
        y = acc / l
        gcol = jnp.sum(jnp.where(lane == hp * 2 + sub, g, 0.0), axis=1, keepdims=True)
        y_ref[:, sub * DH:(sub + 1) * DH] = y * gcol


def _out_kernel(y_ref, wo_ref, o_ref):
    o_ref[...] = _dot(y_ref[...], wo_ref[...], (((1,), (0,)), ((), ())))


def _rope_tables():
    inv_freq = 1.0 / (ROPE_BASE ** (np.arange(0, DH, 2, dtype=np.float64) / DH))
    ang = np.arange(T, dtype=np.float64)[:, None] * inv_freq[None, :]
    cs = np.zeros((T, 128), dtype=np.float32)
    cs[:, 0:32] = np.cos(ang)
    cs[:, 32:64] = np.sin(ang)
    return jnp.asarray(cs)


def _deinterleave_perm():
    perm = np.zeros(D, dtype=np.int32)
    for h in range(H):
        base = h * DH
        perm[base:base + 32] = base + 2 * np.arange(32)
        perm[base + 32:base + DH] = base + 2 * np.arange(32) + 1
    return perm


def kernel(x, W_q, W_k, W_v, W_o, W_router):
    x2 = x.reshape(T, D)
    perm = _deinterleave_perm()
    W_qp = W_q[:, perm]
    W_kp = W_k[:, perm]
    W_r = jnp.pad(W_router, ((0, 0), (0, 128 - H)))
    cs = _rope_tables()

    nb = T // BT
    q, k, v, g = pl.pallas_call(
        _proj_kernel,
        grid=(nb,),
        in_specs=[
            pl.BlockSpec((BT, D), lambda i: (i, 0)),
            pl.BlockSpec((D, D), lambda i: (0, 0)),
            pl.BlockSpec((D, D), lambda i: (0, 0)),
            pl.BlockSpec((D, D), lambda i: (0, 0)),
            pl.BlockSpec((D, 128), lambda i: (0, 0)),
            pl.BlockSpec((BT, 128), lambda i: (i, 0)),
        ],
        out_specs=[
            pl.BlockSpec((BT, D), lambda i: (i, 0)),
            pl.BlockSpec((BT, D), lambda i: (i, 0)),
            pl.BlockSpec((BT, D), lambda i: (i, 0)),
            pl.BlockSpec((BT, 128), lambda i: (i, 0)),
        ],
        out_shape=[
            jax.ShapeDtypeStruct((T, D), jnp.float32),
            jax.ShapeDtypeStruct((T, D), jnp.float32),
            jax.ShapeDtypeStruct((T, D), jnp.float32),
            jax.ShapeDtypeStruct((T, 128), jnp.float32),
        ],
    )(x2, W_qp, W_kp, W_v, W_r, cs)

    y = pl.pallas_call(
        _attn_kernel,
        grid=(H // 2, T // BQ),
        in_specs=[
            pl.BlockSpec((BQ, 128), lambda hp, i: (i, hp)),
            pl.BlockSpec((T, 128), lambda hp, i: (0, hp)),
            pl.BlockSpec((T, 128), lambda hp, i: (0, hp)),
            pl.BlockSpec((BQ, 128), lambda hp, i: (i, 0)),
        ],
        out_specs=pl.BlockSpec((BQ, 128), lambda hp, i: (i, hp)),
        out_shape=jax.ShapeDtypeStruct((T, D), jnp.float32),
    )(q, k, v, g)

    out = pl.pallas_call(
        _out_kernel,
        grid=(nb,),
        in_specs=[
            pl.BlockSpec((BT, D), lambda i: (i, 0)),
            pl.BlockSpec((D, D), lambda i: (0, 0)),
        ],
        out_specs=pl.BlockSpec((BT, D), lambda i: (i, 0)),
        out_shape=jax.ShapeDtypeStruct((T, D), jnp.float32),
    )(y, W_o)

    return out.reshape(1, T, D)


# speedup vs baseline: 2.0620x; 2.0620x over previous
import functools
import numpy as np
import jax
import jax.numpy as jnp
from jax.experimental import pallas as pl

T = 2048
D = 1024
H = 16
DH = 64
K_ACT = 4
ROPE_BASE = 10000.0
BT = 256
BQ = 256
BKV = 256
NEG = -1e30

_dot = functools.partial(jax.lax.dot_general, preferred_element_type=jnp.float32)


def _proj_kernel(x_ref, wq_ref, wk_ref, wv_ref, wr_ref, cs_ref, q_ref, k_ref, v_ref, g_ref):
    x = x_ref[...]
    q = _dot(x, wq_ref[...], (((1,), (0,)), ((), ())))
    k = _dot(x, wk_ref[...], (((1,), (0,)), ((), ())))
    v_ref[...] = _dot(x, wv_ref[...], (((1,), (0,)), ((), ())))

    c = cs_ref[:, 0:32]
    s = cs_ref[:, 32:64]
    q_parts = []
    k_parts = []
    for h in range(H):
        q1 = q[:, h * DH:h * DH + 32]
        q2 = q[:, h * DH + 32:(h + 1) * DH]
        k1 = k[:, h * DH:h * DH + 32]
        k2 = k[:, h * DH + 32:(h + 1) * DH]
        q_parts.append(q1 * c - q2 * s)
        q_parts.append(q1 * s + q2 * c)
        k_parts.append(k1 * c - k2 * s)
        k_parts.append(k1 * s + k2 * c)
    q_ref[...] = jnp.concatenate(q_parts, axis=1)
    k_ref[...] = jnp.concatenate(k_parts, axis=1)

    logits = _dot(x, wr_ref[...], (((1,), (0,)), ((), ())))
    lane = jax.lax.broadcasted_iota(jnp.int32, (BT, 128), 1)
    valid = lane < H
    logits = jnp.where(valid, logits, NEG)
    m = jnp.max(logits, axis=-1, keepdims=True)
    e = jnp.where(valid, jnp.exp(logits - m), 0.0)
    p = e / jnp.sum(e, axis=-1, keepdims=True)

    avail = valid
    gates = jnp.zeros_like(p)
    tot = jnp.zeros((BT, 1), dtype=jnp.float32)
    for _ in range(K_ACT):
        pm = jnp.where(avail, p, -1.0)
        mx = jnp.max(pm, axis=-1, keepdims=True)
        first = jnp.min(jnp.where(pm == mx, lane, 9999), axis=-1, keepdims=True)
        sel = lane == first
        gates = gates + jnp.where(sel, p, 0.0)
        tot = tot + mx
        avail = jnp.logical_and(avail, jnp.logical_not(sel))
    g_ref[...] = gates / (tot + 1e-9)


def _attn_kernel(L, ibase, q_ref, k_ref, v_ref, g_ref, y_ref):
    ii = pl.program_id(1)
    hp = pl.program_id(0)
    scale = jnp.float32(1.0 / np.sqrt(DH))
    row = jax.lax.broadcasted_iota(jnp.int32, (BQ, L), 0) + (ibase + ii) * BQ
    col = jax.lax.broadcasted_iota(jnp.int32, (BQ, L), 1)
    mask = col <= row
    lane = jax.lax.broadcasted_iota(jnp.int32, (BQ, 128), 1)
    g = g_ref[...]
    q0 = q_ref[:, 0:DH]
    q1 = q_ref[:, DH:2 * DH]
    k0 = k_ref[:, 0:DH]
    k1 = k_ref[:, DH:2 * DH]
    s0 = _dot(q0, k0, (((1,), (1,)), ((), ())))
    s1 = _dot(q1, k1, (((1,), (1,)), ((), ())))
    s0 = jnp.where(mask, s0 * scale, NEG)
    s1 = jnp.where(mask, s1 * scale, NEG)
    m0 = jnp.max(s0, axis=-1, keepdims=True)
    m1 = jnp.max(s1, axis=-1, keepdims=True)
    e0 = jnp.exp(s0 - m0)
    e1 = jnp.exp(s1 - m1)
    a0 = e0 / jnp.sum(e0, axis=-1, keepdims=True)
    a1 = e1 / jnp.sum(e1, axis=-1, keepdims=True)
    y0 = _dot(a0, v_ref[:, 0:DH], (((1,), (0,)), ((), ())))
    y1 = _dot(a1, v_ref[:, DH:2 * DH], (((1,), (0,)), ((), ())))
    g0 = jnp.sum(jnp.where(lane == hp * 2, g, 0.0), axis=1, keepdims=True)
    g1 = jnp.sum(jnp.where(lane == hp * 2 + 1, g, 0.0), axis=1, keepdims=True)
    y_ref[...] = jnp.concatenate([y0 * g0, y1 * g1], axis=1)


def _out_kernel(y_ref, wo_ref, o_ref):
    o_ref[...] = _dot(y_ref[...], wo_ref[...], (((1,), (0,)), ((), ())))


def _rope_tables():
    inv_freq = 1.0 / (ROPE_BASE ** (np.arange(0, DH, 2, dtype=np.float64) / DH))
    ang = np.arange(T, dtype=np.float64)[:, None] * inv_freq[None, :]
    cs = np.zeros((T, 128), dtype=np.float32)
    cs[:, 0:32] = np.cos(ang)
    cs[:, 32:64] = np.sin(ang)
    return jnp.asarray(cs)


def _deinterleave_perm():
    perm = np.zeros(D, dtype=np.int32)
    for h in range(H):
        base = h * DH
        perm[base:base + 32] = base + 2 * np.arange(32)
        perm[base + 32:base + DH] = base + 2 * np.arange(32) + 1
    return perm


def kernel(x, W_q, W_k, W_v, W_o, W_router):
    x2 = x.reshape(T, D)
    perm = _deinterleave_perm()
    W_qp = W_q[:, perm]
    W_kp = W_k[:, perm]
    W_r = jnp.pad(W_router, ((0, 0), (0, 128 - H)))
    cs = _rope_tables()

    nb = T // BT
    q, k, v, g = pl.pallas_call(
        _proj_kernel,
        grid=(nb,),
        in_specs=[
            pl.BlockSpec((BT, D), lambda i: (i, 0)),
            pl.BlockSpec((D, D), lambda i: (0, 0)),
            pl.BlockSpec((D, D), lambda i: (0, 0)),
            pl.BlockSpec((D, D), lambda i: (0, 0)),
            pl.BlockSpec((D, 128), lambda i: (0, 0)),
            pl.BlockSpec((BT, 128), lambda i: (i, 0)),
        ],
        out_specs=[
            pl.BlockSpec((BT, D), lambda i: (i, 0)),
            pl.BlockSpec((BT, D), lambda i: (i, 0)),
            pl.BlockSpec((BT, D), lambda i: (i, 0)),
            pl.BlockSpec((BT, 128), lambda i: (i, 0)),
        ],
        out_shape=[
            jax.ShapeDtypeStruct((T, D), jnp.float32),
            jax.ShapeDtypeStruct((T, D), jnp.float32),
            jax.ShapeDtypeStruct((T, D), jnp.float32),
            jax.ShapeDtypeStruct((T, 128), jnp.float32),
        ],
    )(x2, W_qp, W_kp, W_v, W_r, cs)

    y_parts = []
    for grp in range(4):
        L = 512 * (grp + 1)
        ibase = 2 * grp
        y_parts.append(pl.pallas_call(
            functools.partial(_attn_kernel, L, ibase),
            grid=(H // 2, 2),
            in_specs=[
                pl.BlockSpec((BQ, 128), lambda hp, ii, ibase=ibase: (ibase + ii, hp)),
                pl.BlockSpec((L, 128), lambda hp, ii: (0, hp)),
                pl.BlockSpec((L, 128), lambda hp, ii: (0, hp)),
                pl.BlockSpec((BQ, 128), lambda hp, ii, ibase=ibase: (ibase + ii, 0)),
            ],
            out_specs=pl.BlockSpec((BQ, 128), lambda hp, ii: (ii, hp)),
            out_shape=jax.ShapeDtypeStruct((2 * BQ, D), jnp.float32),
        )(q, k, v, g))
    y = jnp.concatenate(y_parts, axis=0)

    out = pl.pallas_call(
        _out_kernel,
        grid=(nb,),
        in_specs=[
            pl.BlockSpec((BT, D), lambda i: (i, 0)),
            pl.BlockSpec((D, D), lambda i: (0, 0)),
        ],
        out_specs=pl.BlockSpec((BT, D), lambda i: (i, 0)),
        out_shape=jax.ShapeDtypeStruct((T, D), jnp.float32),
    )(y, W_o)

    return out.reshape(1, T, D)


# speedup vs baseline: 2.0670x; 1.0024x over previous
import functools
import numpy as np
import jax
import jax.numpy as jnp
from jax.experimental import pallas as pl
from jax.experimental.pallas import tpu as pltpu

T = 2048
D = 1024
H = 16
DH = 64
K_ACT = 4
ROPE_BASE = 10000.0
BT = 256
BQ = 256
BKV = 256
NEG = -1e30

_dot = functools.partial(jax.lax.dot_general, preferred_element_type=jnp.float32)


def _proj_kernel(x_ref, wq_ref, wk_ref, wv_ref, wr_ref, cs_ref, sn_ref, q_ref, k_ref, v_ref, g_ref):
    x = x_ref[...]
    q = _dot(x, wq_ref[...], (((1,), (0,)), ((), ())))
    k = _dot(x, wk_ref[...], (((1,), (0,)), ((), ())))
    v_ref[...] = _dot(x, wv_ref[...], (((1,), (0,)), ((), ())))

    csf = cs_ref[...]
    snf = sn_ref[...]
    lane_d = jax.lax.broadcasted_iota(jnp.int32, (BT, D), 1)
    firsthalf = (lane_d & 63) < 32
    q_sh = jnp.where(firsthalf, pltpu.roll(q, D - 32, 1), pltpu.roll(q, 32, 1))
    k_sh = jnp.where(firsthalf, pltpu.roll(k, D - 32, 1), pltpu.roll(k, 32, 1))
    q_ref[...] = q * csf + q_sh * snf
    k_ref[...] = k * csf + k_sh * snf

    logits = _dot(x, wr_ref[...], (((1,), (0,)), ((), ())))
    lane = jax.lax.broadcasted_iota(jnp.int32, (BT, 128), 1)
    valid = lane < H
    logits = jnp.where(valid, logits, NEG)
    m = jnp.max(logits, axis=-1, keepdims=True)
    e = jnp.where(valid, jnp.exp(logits - m), 0.0)
    p = e / jnp.sum(e, axis=-1, keepdims=True)

    avail = valid
    gates = jnp.zeros_like(p)
    tot = jnp.zeros((BT, 1), dtype=jnp.float32)
    for _ in range(K_ACT):
        pm = jnp.where(avail, p, -1.0)
        mx = jnp.max(pm, axis=-1, keepdims=True)
        first = jnp.min(jnp.where(pm == mx, lane, 9999), axis=-1, keepdims=True)
        sel = lane == first
        gates = gates + jnp.where(sel, p, 0.0)
        tot = tot + mx
        avail = jnp.logical_and(avail, jnp.logical_not(sel))
    g_ref[...] = gates / (tot + 1e-9)


def _attn_kernel(L, ibase, q_ref, k_ref, v_ref, g_ref, y_ref):
    ii = pl.program_id(1)
    hp = pl.program_id(0)
    scale = jnp.float32(1.0 / np.sqrt(DH))
    row = jax.lax.broadcasted_iota(jnp.int32, (BQ, L), 0) + (ibase + ii) * BQ
    col = jax.lax.broadcasted_iota(jnp.int32, (BQ, L), 1)
    mask = col <= row
    lane = jax.lax.broadcasted_iota(jnp.int32, (BQ, 128), 1)
    g = g_ref[...]
    q0 = q_ref[:, 0:DH]
    q1 = q_ref[:, DH:2 * DH]
    k0 = k_ref[:, 0:DH]
    k1 = k_ref[:, DH:2 * DH]
    s0 = _dot(q0, k0, (((1,), (1,)), ((), ())))
    s1 = _dot(q1, k1, (((1,), (1,)), ((), ())))
    s0 = jnp.where(mask, s0 * scale, NEG)
    s1 = jnp.where(mask, s1 * scale, NEG)
    m0 = jnp.max(s0, axis=-1, keepdims=True)
    m1 = jnp.max(s1, axis=-1, keepdims=True)
    e0 = jnp.exp(s0 - m0)
    e1 = jnp.exp(s1 - m1)
    a0 = e0 / jnp.sum(e0, axis=-1, keepdims=True)
    a1 = e1 / jnp.sum(e1, axis=-1, keepdims=True)
    y0 = _dot(a0, v_ref[:, 0:DH], (((1,), (0,)), ((), ())))
    y1 = _dot(a1, v_ref[:, DH:2 * DH], (((1,), (0,)), ((), ())))
    g0 = jnp.sum(jnp.where(lane == hp * 2, g, 0.0), axis=1, keepdims=True)
    g1 = jnp.sum(jnp.where(lane == hp * 2 + 1, g, 0.0), axis=1, keepdims=True)
    y_ref[...] = jnp.concatenate([y0 * g0, y1 * g1], axis=1)


def _out_kernel(y_ref, wo_ref, o_ref):
    o_ref[...] = _dot(y_ref[...], wo_ref[...], (((1,), (0,)), ((), ())))


def _rope_tables():
    inv_freq = 1.0 / (ROPE_BASE ** (np.arange(0, DH, 2, dtype=np.float64) / DH))
    ang = np.arange(T, dtype=np.float64)[:, None] * inv_freq[None, :]
    c = np.cos(ang).astype(np.float32)
    s = np.sin(ang).astype(np.float32)
    half = np.concatenate([c, c], axis=1)
    cs_full = np.tile(half, (1, H))
    sn_half = np.concatenate([-s, s], axis=1)
    sn_full = np.tile(sn_half, (1, H))
    return jnp.asarray(cs_full), jnp.asarray(sn_full)


def _deinterleave_perm():
    perm = np.zeros(D, dtype=np.int32)
    for h in range(H):
        base = h * DH
        perm[base:base + 32] = base + 2 * np.arange(32)
        perm[base + 32:base + DH] = base + 2 * np.arange(32) + 1
    return perm


def kernel(x, W_q, W_k, W_v, W_o, W_router):
    x2 = x.reshape(T, D)
    perm = _deinterleave_perm()
    W_qp = W_q[:, perm]
    W_kp = W_k[:, perm]
    W_r = jnp.pad(W_router, ((0, 0), (0, 128 - H)))
    cs, sn = _rope_tables()

    nb = T // BT
    q, k, v, g = pl.pallas_call(
        _proj_kernel,
        grid=(nb,),
        in_specs=[
            pl.BlockSpec((BT, D), lambda i: (i, 0)),
            pl.BlockSpec((D, D), lambda i: (0, 0)),
            pl.BlockSpec((D, D), lambda i: (0, 0)),
            pl.BlockSpec((D, D), lambda i: (0, 0)),
            pl.BlockSpec((D, 128), lambda i: (0, 0)),
            pl.BlockSpec((BT, D), lambda i: (i, 0)),
            pl.BlockSpec((BT, D), lambda i: (i, 0)),
        ],
        out_specs=[
            pl.BlockSpec((BT, D), lambda i: (i, 0)),
            pl.BlockSpec((BT, D), lambda i: (i, 0)),
            pl.BlockSpec((BT, D), lambda i: (i, 0)),
            pl.BlockSpec((BT, 128), lambda i: (i, 0)),
        ],
        out_shape=[
            jax.ShapeDtypeStruct((T, D), jnp.float32),
            jax.ShapeDtypeStruct((T, D), jnp.float32),
            jax.ShapeDtypeStruct((T, D), jnp.float32),
            jax.ShapeDtypeStruct((T, 128), jnp.float32),
        ],
    )(x2, W_qp, W_kp, W_v, W_r, cs, sn)

    y_parts = []
    for grp in range(4):
        L = 512 * (grp + 1)
        ibase = 2 * grp
        y_parts.append(pl.pallas_call(
            functools.partial(_attn_kernel, L, ibase),
            grid=(H // 2, 2),
            in_specs=[
                pl.BlockSpec((BQ, 128), lambda hp, ii, ibase=ibase: (ibase + ii, hp)),
                pl.BlockSpec((L, 128), lambda hp, ii: (0, hp)),
                pl.BlockSpec((L, 128), lambda hp, ii: (0, hp)),
                pl.BlockSpec((BQ, 128), lambda hp, ii, ibase=ibase: (ibase + ii, 0)),
            ],
            out_specs=pl.BlockSpec((BQ, 128), lambda hp, ii: (ii, hp)),
            out_shape=jax.ShapeDtypeStruct((2 * BQ, D), jnp.float32),
        )(q, k, v, g))
    y = jnp.concatenate(y_parts, axis=0)

    out = pl.pallas_call(
        _out_kernel,
        grid=(nb,),
        in_specs=[
            pl.BlockSpec((BT, D), lambda i: (i, 0)),
            pl.BlockSpec((D, D), lambda i: (0, 0)),
        ],
        out_specs=pl.BlockSpec((BT, D), lambda i: (i, 0)),
        out_shape=jax.ShapeDtypeStruct((T, D), jnp.float32),
    )(y, W_o)

    return out.reshape(1, T, D)


# speedup vs baseline: 2.2850x; 1.1055x over previous
import functools
import numpy as np
import jax
import jax.numpy as jnp
from jax.experimental import pallas as pl
from jax.experimental.pallas import tpu as pltpu

T = 2048
D = 1024
H = 16
DH = 64
K_ACT = 4
ROPE_BASE = 10000.0
BT = 256
BQ = 256
BKV = 256
NEG = -1e30

_dot = functools.partial(jax.lax.dot_general, preferred_element_type=jnp.float32)


def _proj_kernel(x_ref, wq_ref, wk_ref, wv_ref, wr_ref, cs_ref, sn_ref, q_ref, k_ref, v_ref, g_ref):
    x = x_ref[...]
    q = _dot(x, wq_ref[...], (((1,), (0,)), ((), ())))
    k = _dot(x, wk_ref[...], (((1,), (0,)), ((), ())))
    v_ref[...] = _dot(x, wv_ref[...], (((1,), (0,)), ((), ())))

    csf = cs_ref[...]
    snf = sn_ref[...]
    lane_d = jax.lax.broadcasted_iota(jnp.int32, (BT, D), 1)
    firsthalf = (lane_d & 63) < 32
    q_sh = jnp.where(firsthalf, pltpu.roll(q, D - 32, 1), pltpu.roll(q, 32, 1))
    k_sh = jnp.where(firsthalf, pltpu.roll(k, D - 32, 1), pltpu.roll(k, 32, 1))
    q_ref[...] = q * csf + q_sh * snf
    k_ref[...] = k * csf + k_sh * snf

    logits = _dot(x, wr_ref[...], (((1,), (0,)), ((), ())))
    lane = jax.lax.broadcasted_iota(jnp.int32, (BT, 128), 1)
    valid = lane < H
    logits = jnp.where(valid, logits, NEG)
    m = jnp.max(logits, axis=-1, keepdims=True)
    e = jnp.where(valid, jnp.exp(logits - m), 0.0)
    p = e / jnp.sum(e, axis=-1, keepdims=True)

    avail = valid
    gates = jnp.zeros_like(p)
    tot = jnp.zeros((BT, 1), dtype=jnp.float32)
    for _ in range(K_ACT):
        pm = jnp.where(avail, p, -1.0)
        mx = jnp.max(pm, axis=-1, keepdims=True)
        first = jnp.min(jnp.where(pm == mx, lane, 9999), axis=-1, keepdims=True)
        sel = lane == first
        gates = gates + jnp.where(sel, p, 0.0)
        tot = tot + mx
        avail = jnp.logical_and(avail, jnp.logical_not(sel))
    g_ref[...] = gates / (tot + 1e-9)


def _attn_kernel(L, ibase, q_ref, k_ref, v_ref, g_ref, y_ref):
    ii = pl.program_id(1)
    hp = pl.program_id(0)
    scale = jnp.float32(1.0 / np.sqrt(DH))
    row = jax.lax.broadcasted_iota(jnp.int32, (BQ, L), 0) + (ibase + ii) * BQ
    col = jax.lax.broadcasted_iota(jnp.int32, (BQ, L), 1)
    mask = col <= row
    lane = jax.lax.broadcasted_iota(jnp.int32, (BQ, 128), 1)
    g = g_ref[...]
    qb = q_ref[...].astype(jnp.bfloat16)
    kb = k_ref[...].astype(jnp.bfloat16)
    vb = v_ref[...].astype(jnp.bfloat16)
    q0 = qb[:, 0:DH]
    q1 = qb[:, DH:2 * DH]
    k0 = kb[:, 0:DH]
    k1 = kb[:, DH:2 * DH]
    s0 = _dot(q0, k0, (((1,), (1,)), ((), ())))
    s1 = _dot(q1, k1, (((1,), (1,)), ((), ())))
    s0 = jnp.where(mask, s0 * scale, NEG)
    s1 = jnp.where(mask, s1 * scale, NEG)
    m0 = jnp.max(s0, axis=-1, keepdims=True)
    m1 = jnp.max(s1, axis=-1, keepdims=True)
    e0 = jnp.exp(s0 - m0)
    e1 = jnp.exp(s1 - m1)
    a0 = (e0 / jnp.sum(e0, axis=-1, keepdims=True)).astype(jnp.bfloat16)
    a1 = (e1 / jnp.sum(e1, axis=-1, keepdims=True)).astype(jnp.bfloat16)
    y0 = _dot(a0, vb[:, 0:DH], (((1,), (0,)), ((), ())))
    y1 = _dot(a1, vb[:, DH:2 * DH], (((1,), (0,)), ((), ())))
    g0 = jnp.sum(jnp.where(lane == hp * 2, g, 0.0), axis=1, keepdims=True)
    g1 = jnp.sum(jnp.where(lane == hp * 2 + 1, g, 0.0), axis=1, keepdims=True)
    y_ref[...] = jnp.concatenate([y0 * g0, y1 * g1], axis=1)


def _out_kernel(y_ref, wo_ref, o_ref):
    o_ref[...] = _dot(y_ref[...], wo_ref[...], (((1,), (0,)), ((), ())))


def _rope_tables():
    inv_freq = 1.0 / (ROPE_BASE ** (np.arange(0, DH, 2, dtype=np.float64) / DH))
    ang = np.arange(T, dtype=np.float64)[:, None] * inv_freq[None, :]
    c = np.cos(ang).astype(np.float32)
    s = np.sin(ang).astype(np.float32)
    half = np.concatenate([c, c], axis=1)
    cs_full = np.tile(half, (1, H))
    sn_half = np.concatenate([-s, s], axis=1)
    sn_full = np.tile(sn_half, (1, H))
    return jnp.asarray(cs_full), jnp.asarray(sn_full)


def _deinterleave_perm():
    perm = np.zeros(D, dtype=np.int32)
    for h in range(H):
        base = h * DH
        perm[base:base + 32] = base + 2 * np.arange(32)
        perm[base + 32:base + DH] = base + 2 * np.arange(32) + 1
    return perm


def kernel(x, W_q, W_k, W_v, W_o, W_router):
    x2 = x.reshape(T, D)
    perm = _deinterleave_perm()
    W_qp = W_q[:, perm]
    W_kp = W_k[:, perm]
    W_r = jnp.pad(W_router, ((0, 0), (0, 128 - H)))
    cs, sn = _rope_tables()

    nb = T // BT
    q, k, v, g = pl.pallas_call(
        _proj_kernel,
        grid=(nb,),
        in_specs=[
            pl.BlockSpec((BT, D), lambda i: (i, 0)),
            pl.BlockSpec((D, D), lambda i: (0, 0)),
            pl.BlockSpec((D, D), lambda i: (0, 0)),
            pl.BlockSpec((D, D), lambda i: (0, 0)),
            pl.BlockSpec((D, 128), lambda i: (0, 0)),
            pl.BlockSpec((BT, D), lambda i: (i, 0)),
            pl.BlockSpec((BT, D), lambda i: (i, 0)),
        ],
        out_specs=[
            pl.BlockSpec((BT, D), lambda i: (i, 0)),
            pl.BlockSpec((BT, D), lambda i: (i, 0)),
            pl.BlockSpec((BT, D), lambda i: (i, 0)),
            pl.BlockSpec((BT, 128), lambda i: (i, 0)),
        ],
        out_shape=[
            jax.ShapeDtypeStruct((T, D), jnp.float32),
            jax.ShapeDtypeStruct((T, D), jnp.float32),
            jax.ShapeDtypeStruct((T, D), jnp.float32),
            jax.ShapeDtypeStruct((T, 128), jnp.float32),
        ],
    )(x2, W_qp, W_kp, W_v, W_r, cs, sn)

    y_parts = []
    for grp in range(4):
        L = 512 * (grp + 1)
        ibase = 2 * grp
        y_parts.append(pl.pallas_call(
            functools.partial(_attn_kernel, L, ibase),
            grid=(H // 2, 2),
            in_specs=[
                pl.BlockSpec((BQ, 128), lambda hp, ii, ibase=ibase: (ibase + ii, hp)),
                pl.BlockSpec((L, 128), lambda hp, ii: (0, hp)),
                pl.BlockSpec((L, 128), lambda hp, ii: (0, hp)),
                pl.BlockSpec((BQ, 128), lambda hp, ii, ibase=ibase: (ibase + ii, 0)),
            ],
            out_specs=pl.BlockSpec((BQ, 128), lambda hp, ii: (ii, hp)),
            out_shape=jax.ShapeDtypeStruct((2 * BQ, D), jnp.float32),
        )(q, k, v, g))
    y = jnp.concatenate(y_parts, axis=0)

    out = pl.pallas_call(
        _out_kernel,
        grid=(nb,),
        in_specs=[
            pl.BlockSpec((BT, D), lambda i: (i, 0)),
            pl.BlockSpec((D, D), lambda i: (0, 0)),
        ],
        out_specs=pl.BlockSpec((BT, D), lambda i: (i, 0)),
        out_shape=jax.ShapeDtypeStruct((T, D), jnp.float32),
    )(y, W_o)

    return out.reshape(1, T, D)
